# baseline (speedup 1.0000x reference)
import functools

import jax
import jax.numpy as jnp
from jax import lax
from jax.experimental import pallas as pl
from jax.experimental.pallas import tpu as pltpu
from jax.experimental.pallas import tpu_sc as plsc

BS = 64
C2 = 18
C4 = 36
FH = 37
FW = 62
HW = FH * FW
NP = C2 * HW
NR = C4 * HW
GCOL = 4 * FW
PG = 4 * FW * C2
RG = 4 * FW * C4
NG = FH // 4
TAILP = FW * C2
TAILR = FW * C4


def _build_table(tbl, cmod, nvec):
    c0 = lax.iota(jnp.int32, 16)
    q0 = jnp.full((16,), 0, jnp.int32)
    cmodv = jnp.full((16,), cmod, jnp.int32)
    hwv = jnp.full((16,), HW, jnp.int32)
    one = jnp.full((16,), 1, jnp.int32)
    sixteen = jnp.full((16,), 16, jnp.int32)

    def body(i, carry):
        c, q = carry
        tbl[pl.ds(i * 16, 16)] = c * hwv + q
        cn = c + sixteen
        wrap = cn >= cmodv
        cn = jnp.where(wrap, cn - cmodv, cn)
        qn = jnp.where(wrap, q + one, q)
        return (cn, qn)

    lax.fori_loop(0, nvec, body, (c0, q0), unroll=4)


def _permute_one(src_hbm, dst_hbm, tbl, glen, tail_len, nwords, b,
                 ibuf, obuf):
    pltpu.sync_copy(src_hbm.at[b], ibuf.at[pl.ds(0, nwords)])
    for g in range(NG):
        off = jnp.full((16,), g * GCOL, jnp.int32)

        def body(i, carry):
            idx = tbl[pl.ds(i * 16, 16)] + off
            obuf[pl.ds(i * 16, 16)] = plsc.load_gather(ibuf, [idx])
            return carry

        lax.fori_loop(0, glen // 16, body, 0, unroll=8)
        pltpu.sync_copy(obuf.at[pl.ds(0, glen)],
                        dst_hbm.at[b, pl.ds(g * glen, glen)])
    offt = jnp.full((16,), NG * GCOL, jnp.int32)
    limit = jnp.full((16,), nwords - 1, jnp.int32)

    def tail(i, carry):
        idx = jnp.minimum(tbl[pl.ds(i * 16, 16)] + offt, limit)
        obuf[pl.ds(i * 16, 16)] = plsc.load_gather(ibuf, [idx])
        return carry

    lax.fori_loop(0, (tail_len + 15) // 16, tail, 0, unroll=8)
    pltpu.sync_copy(obuf.at[pl.ds(0, tail_len)],
                    dst_hbm.at[b, pl.ds(NG * glen, tail_len)])


def _sc_body(p_hbm, r_hbm, po_hbm, ro_hbm, ibuf, obuf, tp, tr):
    wid = lax.axis_index("s") * 2 + lax.axis_index("c")
    _build_table(tp, C2, (PG + 16) // 16)
    _build_table(tr, C4, (RG + 16) // 16)
    for bb in range(2):
        b = wid * 2 + bb
        _permute_one(p_hbm, po_hbm, tp, PG, TAILP, NP, b, ibuf, obuf)
        _permute_one(r_hbm, ro_hbm, tr, RG, TAILR, NR, b, ibuf, obuf)


def kernel(preds, regs):
    p2 = preds.reshape(BS, NP)
    r2 = regs.reshape(BS, NR)
    mesh = plsc.VectorSubcoreMesh(core_axis_name="c", subcore_axis_name="s")
    fn = pl.kernel(
        _sc_body,
        out_type=[
            jax.ShapeDtypeStruct((BS, NP), jnp.float32),
            jax.ShapeDtypeStruct((BS, NR), jnp.float32),
        ],
        mesh=mesh,
        compiler_params=pltpu.CompilerParams(
            use_tc_tiling_on_sc=False, needs_layout_passes=False),
        scratch_types=[
            pltpu.VMEM((NR,), jnp.float32),
            pltpu.VMEM((RG + 16,), jnp.float32),
            pltpu.VMEM((PG + 16,), jnp.int32),
            pltpu.VMEM((RG + 16,), jnp.int32),
        ],
    )
    po, ro = fn(p2, r2)
    return (po.reshape(BS, FH, FW, C2),
            ro.reshape(BS, FH, FW, C4 // 4, 4))


# speedup vs baseline: 5.7555x; 5.7555x over previous
import jax
import jax.numpy as jnp
from jax.experimental import pallas as pl


def _swap_kernel(p_ref, r_ref, po_ref, ro_ref):
    po_ref[...] = p_ref[...].reshape(po_ref.shape)
    ro_ref[...] = r_ref[...].reshape(ro_ref.shape)


def kernel(preds, regs):
    bs, c2, fh, fw = preds.shape
    c4 = regs.shape[1]
    pt = jnp.transpose(preds, (1, 2, 0, 3))
    rt = jnp.transpose(regs, (1, 2, 0, 3))
    qo, qr = pl.pallas_call(
        _swap_kernel,
        grid=(fh,),
        in_specs=[
            pl.BlockSpec((c2, 1, bs, fw), lambda h: (0, h, 0, 0)),
            pl.BlockSpec((c4, 1, bs, fw), lambda h: (0, h, 0, 0)),
        ],
        out_specs=[
            pl.BlockSpec((1, c2, bs, fw), lambda h: (h, 0, 0, 0)),
            pl.BlockSpec((1, c4, bs, fw), lambda h: (h, 0, 0, 0)),
        ],
        out_shape=[
            jax.ShapeDtypeStruct((fh, c2, bs, fw), preds.dtype),
            jax.ShapeDtypeStruct((fh, c4, bs, fw), regs.dtype),
        ],
    )(pt, rt)
    po = jnp.transpose(qo, (2, 0, 3, 1))
    ro = jnp.transpose(qr, (2, 0, 3, 1)).reshape(bs, fh, fw, c4 // 4, 4)
    return po, ro


# speedup vs baseline: 5.9344x; 1.0311x over previous
import jax
import jax.numpy as jnp
from jax.experimental import pallas as pl


def _swap_kernel(x_ref, y_ref):
    y_ref[...] = x_ref[...].reshape(y_ref.shape)


def _major_swap(x):
    c, fh, bs, fw = x.shape
    return pl.pallas_call(
        _swap_kernel,
        grid=(fh,),
        in_specs=[pl.BlockSpec((c, 1, bs, fw), lambda h: (0, h, 0, 0))],
        out_specs=pl.BlockSpec((1, c, bs, fw), lambda h: (h, 0, 0, 0)),
        out_shape=jax.ShapeDtypeStruct((fh, c, bs, fw), x.dtype),
    )(x)


def kernel(preds, regs):
    bs, c2, fh, fw = preds.shape
    c4 = regs.shape[1]
    rt = jnp.transpose(regs, (1, 2, 0, 3))
    pt = jnp.transpose(preds, (1, 2, 0, 3))
    qr = _major_swap(rt)
    qo = _major_swap(pt)
    po = jnp.transpose(qo, (2, 0, 3, 1))
    ro = jnp.transpose(qr, (2, 0, 3, 1)).reshape(bs, fh, fw, c4 // 4, 4)
    return po, ro
